# baseline (speedup 1.0000x reference)
import functools

import jax
import jax.numpy as jnp
from jax import lax
from jax.experimental import pallas as pl
from jax.experimental.pallas import tpu as pltpu
from jax.experimental.pallas import tpu_sc as plsc

_G = 512
_GP = 528
_NW = 32
_R = 200


def _score_body(x_ref, w1_ref, b1_ref, w2_ref, b2_ref, e_ref):
    xb = x_ref[...]
    h = jnp.tanh(
        jnp.dot(xb, w1_ref[...], preferred_element_type=jnp.float32)
        + b1_ref[...]
    )
    s = jnp.dot(h, w2_ref[...], preferred_element_type=jnp.float32)
    bound = jnp.sum(jnp.abs(w2_ref[...])) + jnp.abs(b2_ref[0, 0])
    e = jnp.exp(s + (b2_ref[0, 0] - bound))
    e_ref[...] = jnp.broadcast_to(e, e_ref.shape)


def _scores16(x, W1, b1, W2, b2, tile_rows):
    n, d = x.shape
    dh = W1.shape[1]
    t = n // tile_rows
    return pl.pallas_call(
        _score_body,
        grid=(t,),
        in_specs=[
            pl.BlockSpec((tile_rows, d), lambda i: (i, 0)),
            pl.BlockSpec((d, dh), lambda i: (0, 0)),
            pl.BlockSpec((1, dh), lambda i: (0, 0)),
            pl.BlockSpec((dh, 1), lambda i: (0, 0)),
            pl.BlockSpec((1, 1), lambda i: (0, 0)),
        ],
        out_specs=pl.BlockSpec((tile_rows, 16), lambda i: (i, 0)),
        out_shape=jax.ShapeDtypeStruct((n, 16), jnp.float32),
    )(x, W1, b1.reshape(1, dh), W2, b2.reshape(1, 1))


def _make_sc_segsum(n, d):
    mesh = plsc.VectorSubcoreMesh(core_axis_name="c", subcore_axis_name="s",
                                  num_cores=2)
    nj = d // 16
    splits = [((w * n) // _NW) // 8 * 8 for w in range(_NW)] + [n]

    @functools.partial(
        pl.kernel, mesh=mesh,
        out_type=[
            jax.ShapeDtypeStruct((_NW, _GP * d), jnp.float32),
            jax.ShapeDtypeStruct((_NW, _GP * 16), jnp.float32),
        ],
        scratch_types=[
            pltpu.VMEM((_R * d,), jnp.float32),
            pltpu.VMEM((_R * 16,), jnp.float32),
            pltpu.VMEM((_R * 16,), jnp.int32),
            pltpu.VMEM((_GP * d,), jnp.float32),
            pltpu.VMEM((_GP * 16,), jnp.float32),
            pltpu.VMEM((256,), jnp.int32),
        ],
    )
    def segsum(x_hbm, e_hbm, b_hbm, gr_hbm, outa_hbm, outb_hbm,
               xbuf, ebuf, bbuf, acc, den, gbuf):
        cid = lax.axis_index("c")
        sid = lax.axis_index("s")
        wid = sid * 2 + cid

        pltpu.sync_copy(gr_hbm, gbuf)
        rinfo = gbuf[pl.ds(pl.multiple_of(8 * wid, 8), 16)]
        start = rinfo[0]
        length = rinfo[1]
        nchunks = (length + _R - 1) // _R
        z16 = jnp.zeros((16,), jnp.float32)

        def zero_a(i, carry):
            acc[pl.ds(pl.multiple_of(16 * i, 8), 16)] = z16
            return carry

        lax.fori_loop(0, _GP * d // 16, zero_a, 0)

        def zero_b(i, carry):
            den[pl.ds(pl.multiple_of(16 * i, 8), 16)] = z16
            return carry

        lax.fori_loop(0, _GP, zero_b, 0)

        def chunk_body(k, carry):
            row0_raw = start + k * _R
            row0 = pl.multiple_of(jnp.minimum(row0_raw, n - _R), 8)
            delta = row0_raw - row0
            rows = jnp.minimum(length - k * _R, _R)
            pltpu.sync_copy(x_hbm.at[pl.ds(pl.multiple_of(row0 * d, 8), _R * d)],
                            xbuf)
            pltpu.sync_copy(e_hbm.at[pl.ds(pl.multiple_of(row0 * 16, 8), _R * 16)],
                            ebuf)
            pltpu.sync_copy(b_hbm.at[pl.ds(pl.multiple_of(row0 * 16, 8), _R * 16)],
                            bbuf)

            def row_body(r0, carry2):
                r = r0 + delta
                ev = ebuf[pl.ds(pl.multiple_of(r * 16, 8), 16)]
                sv = bbuf[pl.ds(pl.multiple_of(r * 16, 8), 16)]
                seg = sv[0]
                for j in range(nj):
                    xv = xbuf[pl.ds(pl.multiple_of(r * d + j * 16, 8), 16)]
                    plsc.addupdate(
                        acc.at[pl.ds(pl.multiple_of(seg * d + j * 16, 8), 16)],
                        xv * ev)
                plsc.addupdate(den.at[pl.ds(pl.multiple_of(seg * 16, 8), 16)],
                               ev)
                return carry2

            lax.fori_loop(0, rows, row_body, 0)
            return carry

        lax.fori_loop(0, nchunks, chunk_body, 0)

        pltpu.sync_copy(acc, outa_hbm.at[wid])
        pltpu.sync_copy(den, outb_hbm.at[wid])

    return segsum, splits


def _combine_body(pa_ref, pb_ref, out_ref):
    num = jnp.sum(pa_ref[...], axis=0)
    den = jnp.sum(pb_ref[...], axis=0)[:, 0:1]
    out_ref[...] = num / jnp.where(den == 0.0, 1.0, den)


def _combine(pa, pb, d):
    outw = 64
    return pl.pallas_call(
        _combine_body,
        grid=(_G // outw,),
        in_specs=[
            pl.BlockSpec((_NW, outw, d), lambda b: (0, b, 0)),
            pl.BlockSpec((_NW, outw, 16), lambda b: (0, b, 0)),
        ],
        out_specs=pl.BlockSpec((outw, d), lambda b: (b, 0)),
        out_shape=jax.ShapeDtypeStruct((_G, d), jnp.float32),
    )(pa, pb)


def kernel(x, batch, W1, b1, W2, b2):
    n, d = x.shape
    tile_rows = 2000
    batch = batch.astype(jnp.int32)
    e16 = _scores16(x, W1, b1, W2, b2, tile_rows)
    b16 = jnp.broadcast_to(batch.reshape(n, 1), (n, 16))

    segsum, splits = _make_sc_segsum(n, d)
    starts = jnp.array(splits[:_NW], jnp.int32)
    lengths = jnp.array([splits[w + 1] - splits[w] for w in range(_NW)],
                        jnp.int32)
    ranges = jnp.zeros((256,), jnp.int32)
    ranges = ranges.at[0:8 * _NW:8].set(starts)
    ranges = ranges.at[1:8 * _NW:8].set(lengths)

    pa, pb = segsum(x.reshape(n * d), e16.reshape(n * 16),
                    b16.reshape(n * 16), ranges)
    pa = pa.reshape(_NW, _GP, d)
    pb = pb.reshape(_NW, _GP, 16)
    return _combine(pa, pb, d)


# speedup vs baseline: 9.7198x; 9.7198x over previous
import functools

import jax
import jax.numpy as jnp
from jax import lax
from jax.experimental import pallas as pl
from jax.experimental.pallas import tpu as pltpu

_G = 512
_W = 16
_OUTW = 64


def _pick_tile(n: int) -> int:
    best = 0
    for d in range(8, 5001, 8):
        if n % d == 0:
            best = d
    if best == 0:
        raise ValueError(f"no tile size divides {n}")
    return best


def _body(tile_rows, num_tiles, tfirst_sp, nwin_sp,
          x_ref, brow_ref, w1_ref, b1_ref, w2_ref, b2_ref,
          out_ref, acc_ref, den_ref):
    i = pl.program_id(0)

    @pl.when(i == 0)
    def _():
        acc_ref[...] = jnp.zeros_like(acc_ref)
        den_ref[...] = jnp.zeros_like(den_ref)

    @pl.when(i < num_tiles)
    def _():
        xb = x_ref[...]
        ones_row = jnp.ones((1, tile_rows), jnp.float32)
        ht = lax.dot_general(
            w1_ref[...], xb, (((0,), (1,)), ((), ())),
            preferred_element_type=jnp.float32,
        )
        bb = lax.dot_general(
            b1_ref[...], ones_row, (((0,), (0,)), ((), ())),
            preferred_element_type=jnp.float32,
        )
        ht = jnp.tanh(ht + bb)
        st = lax.dot_general(
            w2_ref[...], ht, (((0,), (0,)), ((), ())),
            preferred_element_type=jnp.float32,
        )
        bound = jnp.sum(jnp.abs(w2_ref[...])) + jnp.abs(b2_ref[0, 0])
        et = jnp.exp(st + (b2_ref[0, 0] - bound))
        bt = brow_ref[0]
        etb = jnp.broadcast_to(et, (_W, tile_rows))
        btb = jnp.broadcast_to(bt, (_W, tile_rows))
        cw = lax.broadcasted_iota(jnp.int32, (_W, tile_rows), 0).astype(jnp.float32)
        g0 = tfirst_sp[i]

        def win(j, carry):
            off = (g0 + j * _W).astype(jnp.float32)
            oet = jnp.where(btb == cw + off, etb, 0.0)
            numw = lax.dot_general(
                oet, xb, (((1,), (0,)), ((), ())),
                preferred_element_type=jnp.float32,
            )
            denw = jnp.sum(oet, axis=1, keepdims=True)
            base = g0 + j * _W
            acc_ref[pl.ds(base, _W), :] += numw
            den_ref[pl.ds(base, _W), :] += denw
            return carry

        lax.fori_loop(0, nwin_sp[i], win, 0)

    @pl.when(i >= num_tiles)
    def _():
        b = i - num_tiles
        a = acc_ref[pl.ds(b * _OUTW, _OUTW), :]
        dd = den_ref[pl.ds(b * _OUTW, _OUTW), :]
        out_ref[...] = a / jnp.where(dd == 0.0, 1.0, dd)


def kernel(x, batch, W1, b1, W2, b2):
    n, d = x.shape
    dh = W1.shape[1]
    tile_rows = _pick_tile(n)
    num_tiles = n // tile_rows
    out_steps = _G // _OUTW
    grid = num_tiles + out_steps

    batch = batch.astype(jnp.int32)
    brow = batch.astype(jnp.float32).reshape(num_tiles, 1, tile_rows)
    tidx = jnp.arange(num_tiles, dtype=jnp.int32) * tile_rows
    tfirst = batch[tidx]
    tlast = batch[tidx + tile_rows - 1]
    nwin = (tlast - tfirst) // _W + 1
    pad = jnp.zeros((out_steps,), jnp.int32)
    tfirst = jnp.concatenate([tfirst, pad])
    nwin = jnp.concatenate([nwin, pad])

    grid_spec = pltpu.PrefetchScalarGridSpec(
        num_scalar_prefetch=2,
        grid=(grid,),
        in_specs=[
            pl.BlockSpec((tile_rows, d),
                         lambda i, tf, nw: (jnp.minimum(i, num_tiles - 1), 0)),
            pl.BlockSpec((1, 1, tile_rows),
                         lambda i, tf, nw: (jnp.minimum(i, num_tiles - 1), 0, 0)),
            pl.BlockSpec((d, dh), lambda i, *_: (0, 0)),
            pl.BlockSpec((1, dh), lambda i, *_: (0, 0)),
            pl.BlockSpec((dh, 1), lambda i, *_: (0, 0)),
            pl.BlockSpec((1, 1), lambda i, *_: (0, 0)),
        ],
        out_specs=pl.BlockSpec(
            (_OUTW, d), lambda i, *_: (jnp.maximum(i - num_tiles, 0), 0)),
        scratch_shapes=[
            pltpu.VMEM((_G + _W, d), jnp.float32),
            pltpu.VMEM((_G + _W, 1), jnp.float32),
        ],
    )
    out = pl.pallas_call(
        functools.partial(_body, tile_rows, num_tiles),
        grid_spec=grid_spec,
        out_shape=jax.ShapeDtypeStruct((_G, d), jnp.float32),
    )(tfirst, nwin,
      x, brow, W1, b1.reshape(1, dh), W2, b2.reshape(1, 1))
    return out
